# baseline (speedup 1.0000x reference)
import functools

import jax
import jax.numpy as jnp
import numpy as np
from jax.experimental import pallas as pl
from jax.experimental.pallas import tpu as pltpu


def _resize_matrix(out_size: int, in_size: int) -> np.ndarray:
    scale = in_size / out_size
    x = (np.arange(out_size) + 0.5) * scale - 0.5
    i0 = np.floor(x).astype(np.int64)
    w = (x - i0).astype(np.float32)
    m = np.zeros((out_size, in_size), np.float32)
    rows = np.arange(out_size)
    np.add.at(m, (rows, np.clip(i0, 0, in_size - 1)), 1.0 - w)
    np.add.at(m, (rows, np.clip(i0 + 1, 0, in_size - 1)), w)
    return m


_R_UP = _resize_matrix(224, 28)
_R_UP_T = np.ascontiguousarray(_R_UP.T)


def _xt_kernel(img_ref, wt_ref, xta_ref, qsq_ref):
    blk = img_ref[...]
    x = blk.reshape(8, 3, 4, 8, 28, 8)
    xt = x.transpose(1, 3, 5, 2, 0, 4).reshape(192, 896)
    xb = xt.astype(jnp.bfloat16)
    xta_ref[0:192, :] = xb
    pad = jnp.concatenate(
        [jnp.ones((2, 896), jnp.bfloat16), jnp.zeros((62, 896), jnp.bfloat16)]
    )
    xta_ref[192:256, :] = pad
    f = jnp.dot(
        wt_ref[...].astype(jnp.bfloat16), xb, preferred_element_type=jnp.float32
    )
    qsq_ref[...] = jnp.sum(f * f, axis=0, keepdims=True)


def _dist_kernel(xta_ref, wt_ref, mb_ref, out_ref, acc_ref):
    k = pl.program_id(0)
    nk = pl.num_programs(0)
    mb = mb_ref[...]
    wn2 = (-2.0 * wt_ref[...]).astype(jnp.bfloat16)
    mbp = jnp.dot(
        mb.astype(jnp.bfloat16), wn2, preferred_element_type=jnp.float32
    )
    ksq = jnp.sum(mb * mb, axis=1, keepdims=True)
    hi = ksq.astype(jnp.bfloat16)
    lo = (ksq - hi.astype(jnp.float32)).astype(jnp.bfloat16)
    kb = mb.shape[0]
    aug = jnp.concatenate(
        [mbp.astype(jnp.bfloat16), hi, lo, jnp.zeros((kb, 62), jnp.bfloat16)],
        axis=1,
    )
    d = jnp.dot(
        aug, xta_ref[...], preferred_element_type=jnp.float32
    )
    part8 = jnp.min(d.reshape(kb // 8, 8, d.shape[1]), axis=0)

    @pl.when(k == 0)
    def _init():
        acc_ref[...] = part8

    @pl.when(k > 0)
    def _acc():
        acc_ref[...] = jnp.minimum(acc_ref[...], part8)

    @pl.when(k == nk - 1)
    def _fin():
        out_ref[...] = jnp.min(acc_ref[...], axis=0, keepdims=True)


def _post_kernel(dmin_ref, qsq_ref, r_ref, rt_ref, up_ref, mx_ref):
    g = dmin_ref[0] + qsq_ref[0]
    t = jnp.dot(r_ref[...], g, preferred_element_type=jnp.float32)
    up_ref[0] = jnp.dot(t, rt_ref[...], preferred_element_type=jnp.float32)
    mx_ref[0] = jnp.broadcast_to(jnp.max(g), (1, 128))


@functools.partial(jax.jit, static_argnames=())
def kernel(batch_images, W, memory_bank):
    B = batch_images.shape[0]
    Q = B * 28 * 28
    F = W.shape[1]
    K = memory_bank.shape[0]

    imgs = batch_images.reshape(B, 3, 28, 8, 224)
    wt = W.T

    QB = 896
    xta, qsq = pl.pallas_call(
        _xt_kernel,
        grid=(Q // QB,),
        in_specs=[
            pl.BlockSpec((B, 3, 4, 8, 224), lambda g: (0, 0, g, 0, 0)),
            pl.BlockSpec((F, 192), lambda g: (0, 0)),
        ],
        out_specs=[
            pl.BlockSpec((256, QB), lambda g: (0, g)),
            pl.BlockSpec((1, QB), lambda g: (0, g)),
        ],
        out_shape=[
            jax.ShapeDtypeStruct((256, Q), jnp.bfloat16),
            jax.ShapeDtypeStruct((1, Q), jnp.float32),
        ],
    )(imgs, wt)

    KB = 1024
    dmin = pl.pallas_call(
        _dist_kernel,
        grid=(K // KB,),
        in_specs=[
            pl.BlockSpec((256, Q), lambda k: (0, 0)),
            pl.BlockSpec((F, 192), lambda k: (0, 0)),
            pl.BlockSpec((KB, F), lambda k: (k, 0)),
        ],
        out_specs=pl.BlockSpec((1, Q), lambda k: (0, 0)),
        out_shape=jax.ShapeDtypeStruct((1, Q), jnp.float32),
        scratch_shapes=[pltpu.VMEM((8, Q), jnp.float32)],
    )(xta, wt, memory_bank)

    dmin_r = dmin.reshape(28, B, 28).transpose(1, 0, 2)
    qsq_r = qsq.reshape(28, B, 28).transpose(1, 0, 2)
    r_up = jnp.asarray(_R_UP)
    r_up_t = jnp.asarray(_R_UP_T)

    ups, mx = pl.pallas_call(
        _post_kernel,
        grid=(B,),
        in_specs=[
            pl.BlockSpec((1, 28, 28), lambda b: (b, 0, 0)),
            pl.BlockSpec((1, 28, 28), lambda b: (b, 0, 0)),
            pl.BlockSpec((224, 28), lambda b: (0, 0)),
            pl.BlockSpec((28, 224), lambda b: (0, 0)),
        ],
        out_specs=[
            pl.BlockSpec((1, 224, 224), lambda b: (b, 0, 0)),
            pl.BlockSpec((1, 1, 128), lambda b: (b, 0, 0)),
        ],
        out_shape=[
            jax.ShapeDtypeStruct((B, 224, 224), jnp.float32),
            jax.ShapeDtypeStruct((B, 1, 128), jnp.float32),
        ],
    )(dmin_r, qsq_r, r_up, r_up_t)

    image_scores = mx[:, 0, 0]
    return image_scores, ups


# speedup vs baseline: 1.3068x; 1.3068x over previous
import functools

import jax
import jax.numpy as jnp
import numpy as np
from jax.experimental import pallas as pl
from jax.experimental.pallas import tpu as pltpu


def _resize_matrix(out_size: int, in_size: int) -> np.ndarray:
    scale = in_size / out_size
    x = (np.arange(out_size) + 0.5) * scale - 0.5
    i0 = np.floor(x).astype(np.int64)
    w = (x - i0).astype(np.float32)
    m = np.zeros((out_size, in_size), np.float32)
    rows = np.arange(out_size)
    np.add.at(m, (rows, np.clip(i0, 0, in_size - 1)), 1.0 - w)
    np.add.at(m, (rows, np.clip(i0 + 1, 0, in_size - 1)), w)
    return m


_R_UP = _resize_matrix(224, 28)
_R_UP_T = np.ascontiguousarray(_R_UP.T)

_E1 = np.zeros((224, 224), np.float32)
for _gw in range(28):
    for _pw in range(8):
        _E1[8 * _gw + _pw, 28 * _pw + _gw] = 1.0


def _xt_kernel(img_ref, wt_ref, e1_ref, xta_ref, qsq_ref):
    blk = img_ref[...].astype(jnp.bfloat16)
    m = blk.reshape(768, 224)
    m1 = jnp.dot(
        m, e1_ref[...].astype(jnp.bfloat16), preferred_element_type=jnp.float32
    ).astype(jnp.bfloat16)
    x = m1.reshape(8, 3, 4, 8, 8, 28)
    xb = x.transpose(1, 3, 4, 2, 0, 5).reshape(192, 896)
    xta_ref[0:192, :] = xb
    pad = jnp.concatenate(
        [jnp.ones((2, 896), jnp.bfloat16), jnp.zeros((62, 896), jnp.bfloat16)]
    )
    xta_ref[192:256, :] = pad
    f = jnp.dot(
        wt_ref[...].astype(jnp.bfloat16), xb, preferred_element_type=jnp.float32
    )
    qsq_ref[...] = jnp.sum(f * f, axis=0, keepdims=True)


def _dist_kernel(xta_ref, wt_ref, mb_ref, out_ref, acc_ref):
    k = pl.program_id(0)
    nk = pl.num_programs(0)
    mb = mb_ref[...]
    wn2 = (-2.0 * wt_ref[...]).astype(jnp.bfloat16)
    mbp = jnp.dot(
        mb.astype(jnp.bfloat16), wn2, preferred_element_type=jnp.float32
    )
    ksq = jnp.sum(mb * mb, axis=1, keepdims=True)
    hi = ksq.astype(jnp.bfloat16)
    lo = (ksq - hi.astype(jnp.float32)).astype(jnp.bfloat16)
    kb = mb.shape[0]
    aug = jnp.concatenate(
        [mbp.astype(jnp.bfloat16), hi, lo, jnp.zeros((kb, 62), jnp.bfloat16)],
        axis=1,
    )
    d = jnp.dot(
        aug, xta_ref[...], preferred_element_type=jnp.float32
    )
    part8 = jnp.min(d.reshape(kb // 8, 8, d.shape[1]), axis=0)

    @pl.when(k == 0)
    def _init():
        acc_ref[...] = part8

    @pl.when(k > 0)
    def _acc():
        acc_ref[...] = jnp.minimum(acc_ref[...], part8)

    @pl.when(k == nk - 1)
    def _fin():
        out_ref[...] = jnp.min(acc_ref[...], axis=0, keepdims=True)


def _post_kernel(dmin_ref, qsq_ref, r_ref, rt_ref, up_ref, mx_ref):
    g = dmin_ref[0] + qsq_ref[0]
    t = jnp.dot(r_ref[...], g, preferred_element_type=jnp.float32)
    up_ref[0] = jnp.dot(t, rt_ref[...], preferred_element_type=jnp.float32)
    mx_ref[0] = jnp.broadcast_to(jnp.max(g), (1, 128))


@functools.partial(jax.jit, static_argnames=())
def kernel(batch_images, W, memory_bank):
    B = batch_images.shape[0]
    Q = B * 28 * 28
    F = W.shape[1]
    K = memory_bank.shape[0]

    imgs = batch_images.reshape(B, 3, 28, 8, 224)
    wt = W.T

    QB = 896
    xta, qsq = pl.pallas_call(
        _xt_kernel,
        grid=(Q // QB,),
        in_specs=[
            pl.BlockSpec((B, 3, 4, 8, 224), lambda g: (0, 0, g, 0, 0)),
            pl.BlockSpec((F, 192), lambda g: (0, 0)),
            pl.BlockSpec((224, 224), lambda g: (0, 0)),
        ],
        out_specs=[
            pl.BlockSpec((256, QB), lambda g: (0, g)),
            pl.BlockSpec((1, QB), lambda g: (0, g)),
        ],
        out_shape=[
            jax.ShapeDtypeStruct((256, Q), jnp.bfloat16),
            jax.ShapeDtypeStruct((1, Q), jnp.float32),
        ],
    )(imgs, wt, jnp.asarray(_E1))

    KB = 1024
    dmin = pl.pallas_call(
        _dist_kernel,
        grid=(K // KB,),
        in_specs=[
            pl.BlockSpec((256, Q), lambda k: (0, 0)),
            pl.BlockSpec((F, 192), lambda k: (0, 0)),
            pl.BlockSpec((KB, F), lambda k: (k, 0)),
        ],
        out_specs=pl.BlockSpec((1, Q), lambda k: (0, 0)),
        out_shape=jax.ShapeDtypeStruct((1, Q), jnp.float32),
        scratch_shapes=[pltpu.VMEM((8, Q), jnp.float32)],
    )(xta, wt, memory_bank)

    dmin_r = dmin.reshape(28, B, 28).transpose(1, 0, 2)
    qsq_r = qsq.reshape(28, B, 28).transpose(1, 0, 2)
    r_up = jnp.asarray(_R_UP)
    r_up_t = jnp.asarray(_R_UP_T)

    ups, mx = pl.pallas_call(
        _post_kernel,
        grid=(B,),
        in_specs=[
            pl.BlockSpec((1, 28, 28), lambda b: (b, 0, 0)),
            pl.BlockSpec((1, 28, 28), lambda b: (b, 0, 0)),
            pl.BlockSpec((224, 28), lambda b: (0, 0)),
            pl.BlockSpec((28, 224), lambda b: (0, 0)),
        ],
        out_specs=[
            pl.BlockSpec((1, 224, 224), lambda b: (b, 0, 0)),
            pl.BlockSpec((1, 1, 128), lambda b: (b, 0, 0)),
        ],
        out_shape=[
            jax.ShapeDtypeStruct((B, 224, 224), jnp.float32),
            jax.ShapeDtypeStruct((B, 1, 128), jnp.float32),
        ],
    )(dmin_r, qsq_r, r_up, r_up_t)

    image_scores = mx[:, 0, 0]
    return image_scores, ups


# speedup vs baseline: 1.3586x; 1.0396x over previous
import functools

import jax
import jax.numpy as jnp
import numpy as np
from jax.experimental import pallas as pl
from jax.experimental.pallas import tpu as pltpu


def _resize_matrix(out_size: int, in_size: int) -> np.ndarray:
    scale = in_size / out_size
    x = (np.arange(out_size) + 0.5) * scale - 0.5
    i0 = np.floor(x).astype(np.int64)
    w = (x - i0).astype(np.float32)
    m = np.zeros((out_size, in_size), np.float32)
    rows = np.arange(out_size)
    np.add.at(m, (rows, np.clip(i0, 0, in_size - 1)), 1.0 - w)
    np.add.at(m, (rows, np.clip(i0 + 1, 0, in_size - 1)), w)
    return m


_R_UP = _resize_matrix(224, 28)
_R_UP_T = np.ascontiguousarray(_R_UP.T)

_E1 = np.zeros((224, 224), np.float32)
for _gw in range(28):
    for _pw in range(8):
        _E1[8 * _gw + _pw, 28 * _pw + _gw] = 1.0


def _xt_prep_kernel(img_ref, wt_ref, e1_ref, mb_ref,
                    xta_ref, qsq_ref, mba_ref):
    g = pl.program_id(0)

    mb = mb_ref[...]
    wn2 = (-2.0 * wt_ref[...]).astype(jnp.bfloat16)
    mbp = jnp.dot(
        mb.astype(jnp.bfloat16), wn2, preferred_element_type=jnp.float32
    )
    ksq = jnp.sum(mb * mb, axis=1, keepdims=True)
    hi = ksq.astype(jnp.bfloat16)
    lo = (ksq - hi.astype(jnp.float32)).astype(jnp.bfloat16)
    kb = mb.shape[0]
    mba_ref[...] = jnp.concatenate(
        [mbp.astype(jnp.bfloat16), hi, lo, jnp.zeros((kb, 62), jnp.bfloat16)],
        axis=1,
    )

    @pl.when(g < 7)
    def _xt():
        blk = img_ref[...].astype(jnp.bfloat16)
        m = blk.reshape(768, 224)
        m1 = jnp.dot(
            m, e1_ref[...].astype(jnp.bfloat16),
            preferred_element_type=jnp.float32,
        ).astype(jnp.bfloat16)
        x = m1.reshape(8, 3, 4, 8, 8, 28)
        xb = x.transpose(1, 3, 4, 2, 0, 5).reshape(192, 896)
        xta_ref[0:192, :] = xb
        pad = jnp.concatenate(
            [jnp.ones((2, 896), jnp.bfloat16),
             jnp.zeros((62, 896), jnp.bfloat16)]
        )
        xta_ref[192:256, :] = pad
        f = jnp.dot(
            wt_ref[...].astype(jnp.bfloat16), xb,
            preferred_element_type=jnp.float32,
        )
        qsq_ref[...] = jnp.sum(f * f, axis=0, keepdims=True)


def _dist_kernel(xta_ref, mba_ref, out_ref, acc_ref):
    k = pl.program_id(0)
    nk = pl.num_programs(0)
    aug = mba_ref[...]
    d = jnp.dot(
        aug, xta_ref[...], preferred_element_type=jnp.float32
    )
    kb = aug.shape[0]
    part8 = jnp.min(d.reshape(kb // 8, 8, d.shape[1]), axis=0)

    @pl.when(k == 0)
    def _init():
        acc_ref[...] = part8

    @pl.when(k > 0)
    def _acc():
        acc_ref[...] = jnp.minimum(acc_ref[...], part8)

    @pl.when(k == nk - 1)
    def _fin():
        out_ref[...] = jnp.min(acc_ref[...], axis=0, keepdims=True)


def _post_kernel(dmin_ref, qsq_ref, r_ref, rt_ref, up_ref, mx_ref):
    g = dmin_ref[0] + qsq_ref[0]
    t = jnp.dot(r_ref[...], g, preferred_element_type=jnp.float32)
    up_ref[0] = jnp.dot(t, rt_ref[...], preferred_element_type=jnp.float32)
    mx_ref[0] = jnp.broadcast_to(jnp.max(g), (1, 128))


@functools.partial(jax.jit, static_argnames=())
def kernel(batch_images, W, memory_bank):
    B = batch_images.shape[0]
    Q = B * 28 * 28
    F = W.shape[1]
    K = memory_bank.shape[0]

    wt = W.T

    QB = 896
    KPB = K // 8
    xta, qsq, mba = pl.pallas_call(
        _xt_prep_kernel,
        grid=(8,),
        in_specs=[
            pl.BlockSpec((B, 3, 32, 224), lambda g: (0, 0, jnp.minimum(g, 6), 0)),
            pl.BlockSpec((F, 192), lambda g: (0, 0)),
            pl.BlockSpec((224, 224), lambda g: (0, 0)),
            pl.BlockSpec((KPB, F), lambda g: (g, 0)),
        ],
        out_specs=[
            pl.BlockSpec((256, QB), lambda g: (0, jnp.minimum(g, 6))),
            pl.BlockSpec((1, QB), lambda g: (0, jnp.minimum(g, 6))),
            pl.BlockSpec((KPB, 256), lambda g: (g, 0)),
        ],
        out_shape=[
            jax.ShapeDtypeStruct((256, Q), jnp.bfloat16),
            jax.ShapeDtypeStruct((1, Q), jnp.float32),
            jax.ShapeDtypeStruct((K, 256), jnp.bfloat16),
        ],
    )(batch_images, wt, jnp.asarray(_E1), memory_bank)

    KB = 1024
    dmin = pl.pallas_call(
        _dist_kernel,
        grid=(K // KB,),
        in_specs=[
            pl.BlockSpec((256, Q), lambda k: (0, 0)),
            pl.BlockSpec((KB, 256), lambda k: (k, 0)),
        ],
        out_specs=pl.BlockSpec((1, Q), lambda k: (0, 0)),
        out_shape=jax.ShapeDtypeStruct((1, Q), jnp.float32),
        scratch_shapes=[pltpu.VMEM((8, Q), jnp.float32)],
    )(xta, mba)

    dmin_r = dmin.reshape(28, B, 28).transpose(1, 0, 2)
    qsq_r = qsq.reshape(28, B, 28).transpose(1, 0, 2)

    ups, mx = pl.pallas_call(
        _post_kernel,
        grid=(B,),
        in_specs=[
            pl.BlockSpec((1, 28, 28), lambda b: (b, 0, 0)),
            pl.BlockSpec((1, 28, 28), lambda b: (b, 0, 0)),
            pl.BlockSpec((224, 28), lambda b: (0, 0)),
            pl.BlockSpec((28, 224), lambda b: (0, 0)),
        ],
        out_specs=[
            pl.BlockSpec((1, 224, 224), lambda b: (b, 0, 0)),
            pl.BlockSpec((1, 1, 128), lambda b: (b, 0, 0)),
        ],
        out_shape=[
            jax.ShapeDtypeStruct((B, 224, 224), jnp.float32),
            jax.ShapeDtypeStruct((B, 1, 128), jnp.float32),
        ],
    )(dmin_r, qsq_r, jnp.asarray(_R_UP), jnp.asarray(_R_UP_T))

    image_scores = mx[:, 0, 0]
    return image_scores, ups


# speedup vs baseline: 1.8130x; 1.3344x over previous
import functools

import jax
import jax.numpy as jnp
import numpy as np
from jax.experimental import pallas as pl
from jax.experimental.pallas import tpu as pltpu


def _resize_matrix(out_size: int, in_size: int) -> np.ndarray:
    scale = in_size / out_size
    x = (np.arange(out_size) + 0.5) * scale - 0.5
    i0 = np.floor(x).astype(np.int64)
    w = (x - i0).astype(np.float32)
    m = np.zeros((out_size, in_size), np.float32)
    rows = np.arange(out_size)
    np.add.at(m, (rows, np.clip(i0, 0, in_size - 1)), 1.0 - w)
    np.add.at(m, (rows, np.clip(i0 + 1, 0, in_size - 1)), w)
    return m


_R_UP = _resize_matrix(224, 28)
_R_UP_T = np.ascontiguousarray(_R_UP.T)

_E1 = np.zeros((224, 224), np.float32)
for _gw in range(28):
    for _pw in range(8):
        _E1[8 * _gw + _pw, 28 * _pw + _gw] = 1.0


def _xt_prep_kernel(img_ref, wt_ref, e1_ref, mb_ref,
                    xta_ref, qsq_ref, mba_ref):
    g = pl.program_id(0)

    mb = mb_ref[...]
    wn2 = (-2.0 * wt_ref[...]).astype(jnp.bfloat16)
    mbp = jnp.dot(
        mb.astype(jnp.bfloat16), wn2, preferred_element_type=jnp.float32
    )
    ksq = jnp.sum(mb * mb, axis=1, keepdims=True)
    f8 = jnp.float8_e4m3fn
    ksq4 = ksq * 0.25
    hi = ksq4.astype(f8)
    r1 = ksq4 - hi.astype(jnp.float32)
    lo1 = r1.astype(f8)
    r2 = r1 - lo1.astype(jnp.float32)
    lo2 = r2.astype(f8)
    lo3 = (r2 - lo2.astype(jnp.float32)).astype(f8)
    kb = mb.shape[0]
    mba_ref[...] = jnp.concatenate(
        [mbp.astype(f8), hi, lo1, lo2, lo3, jnp.zeros((kb, 60), f8)],
        axis=1,
    )

    @pl.when(g < 7)
    def _xt():
        blk = img_ref[...].astype(jnp.bfloat16)
        m = blk.reshape(768, 224)
        m1 = jnp.dot(
            m, e1_ref[...].astype(jnp.bfloat16),
            preferred_element_type=jnp.float32,
        ).astype(jnp.bfloat16)
        x = m1.reshape(8, 3, 4, 8, 8, 28)
        xb = x.transpose(1, 3, 4, 2, 0, 5).reshape(192, 896)
        f8 = jnp.float8_e4m3fn
        xta_ref[0:192, :] = xb.astype(f8)
        pad = jnp.concatenate(
            [jnp.full((4, 896), 4.0, f8), jnp.zeros((60, 896), f8)]
        )
        xta_ref[192:256, :] = pad
        f = jnp.dot(
            wt_ref[...].astype(jnp.bfloat16), xb,
            preferred_element_type=jnp.float32,
        )
        qsq_ref[...] = jnp.sum(f * f, axis=0, keepdims=True)


def _dist_kernel(xta_ref, mba_ref, out_ref, acc_ref):
    k = pl.program_id(0)
    nk = pl.num_programs(0)
    aug = mba_ref[...]
    d = jnp.dot(
        aug, xta_ref[...], preferred_element_type=jnp.float32
    )
    kb = aug.shape[0]
    part8 = jnp.min(d.reshape(kb // 8, 8, d.shape[1]), axis=0)

    @pl.when(k == 0)
    def _init():
        acc_ref[...] = part8

    @pl.when(k > 0)
    def _acc():
        acc_ref[...] = jnp.minimum(acc_ref[...], part8)

    @pl.when(k == nk - 1)
    def _fin():
        out_ref[...] = jnp.min(acc_ref[...], axis=0, keepdims=True)


def _post_kernel(dmin_ref, qsq_ref, r_ref, rt_ref, up_ref, mx_ref):
    g = dmin_ref[0] + qsq_ref[0]
    t = jnp.dot(r_ref[...], g, preferred_element_type=jnp.float32)
    up_ref[0] = jnp.dot(t, rt_ref[...], preferred_element_type=jnp.float32)
    mx_ref[0] = jnp.broadcast_to(jnp.max(g), (1, 128))


@functools.partial(jax.jit, static_argnames=())
def kernel(batch_images, W, memory_bank):
    B = batch_images.shape[0]
    Q = B * 28 * 28
    F = W.shape[1]
    K = memory_bank.shape[0]

    wt = W.T

    QB = 896
    KPB = K // 8
    xta, qsq, mba = pl.pallas_call(
        _xt_prep_kernel,
        grid=(8,),
        in_specs=[
            pl.BlockSpec((B, 3, 32, 224), lambda g: (0, 0, jnp.minimum(g, 6), 0)),
            pl.BlockSpec((F, 192), lambda g: (0, 0)),
            pl.BlockSpec((224, 224), lambda g: (0, 0)),
            pl.BlockSpec((KPB, F), lambda g: (g, 0)),
        ],
        out_specs=[
            pl.BlockSpec((256, QB), lambda g: (0, jnp.minimum(g, 6))),
            pl.BlockSpec((1, QB), lambda g: (0, jnp.minimum(g, 6))),
            pl.BlockSpec((KPB, 256), lambda g: (g, 0)),
        ],
        out_shape=[
            jax.ShapeDtypeStruct((256, Q), jnp.float8_e4m3fn),
            jax.ShapeDtypeStruct((1, Q), jnp.float32),
            jax.ShapeDtypeStruct((K, 256), jnp.float8_e4m3fn),
        ],
    )(batch_images, wt, jnp.asarray(_E1), memory_bank)

    KB = 1024
    dmin = pl.pallas_call(
        _dist_kernel,
        grid=(K // KB,),
        in_specs=[
            pl.BlockSpec((256, Q), lambda k: (0, 0)),
            pl.BlockSpec((KB, 256), lambda k: (k, 0)),
        ],
        out_specs=pl.BlockSpec((1, Q), lambda k: (0, 0)),
        out_shape=jax.ShapeDtypeStruct((1, Q), jnp.float32),
        scratch_shapes=[pltpu.VMEM((8, Q), jnp.float32)],
    )(xta, mba)

    dmin_r = dmin.reshape(28, B, 28).transpose(1, 0, 2)
    qsq_r = qsq.reshape(28, B, 28).transpose(1, 0, 2)

    ups, mx = pl.pallas_call(
        _post_kernel,
        grid=(B,),
        in_specs=[
            pl.BlockSpec((1, 28, 28), lambda b: (b, 0, 0)),
            pl.BlockSpec((1, 28, 28), lambda b: (b, 0, 0)),
            pl.BlockSpec((224, 28), lambda b: (0, 0)),
            pl.BlockSpec((28, 224), lambda b: (0, 0)),
        ],
        out_specs=[
            pl.BlockSpec((1, 224, 224), lambda b: (b, 0, 0)),
            pl.BlockSpec((1, 1, 128), lambda b: (b, 0, 0)),
        ],
        out_shape=[
            jax.ShapeDtypeStruct((B, 224, 224), jnp.float32),
            jax.ShapeDtypeStruct((B, 1, 128), jnp.float32),
        ],
    )(dmin_r, qsq_r, jnp.asarray(_R_UP), jnp.asarray(_R_UP_T))

    image_scores = mx[:, 0, 0]
    return image_scores, ups


# speedup vs baseline: 1.8153x; 1.0013x over previous
import functools

import jax
import jax.numpy as jnp
import numpy as np
from jax.experimental import pallas as pl
from jax.experimental.pallas import tpu as pltpu


def _resize_matrix(out_size: int, in_size: int) -> np.ndarray:
    scale = in_size / out_size
    x = (np.arange(out_size) + 0.5) * scale - 0.5
    i0 = np.floor(x).astype(np.int64)
    w = (x - i0).astype(np.float32)
    m = np.zeros((out_size, in_size), np.float32)
    rows = np.arange(out_size)
    np.add.at(m, (rows, np.clip(i0, 0, in_size - 1)), 1.0 - w)
    np.add.at(m, (rows, np.clip(i0 + 1, 0, in_size - 1)), w)
    return m


_R_UP = _resize_matrix(224, 28)
_R_UP_T = np.ascontiguousarray(_R_UP.T)

_E1 = np.zeros((224, 224), np.float32)
for _gw in range(28):
    for _pw in range(8):
        _E1[8 * _gw + _pw, 28 * _pw + _gw] = 1.0


def _main_kernel(img_ref, wt_ref, e1_ref, mb_ref, qsq_ref, out_ref,
                 xta_s, mba_s, acc_ref):
    g = pl.program_id(0)
    ng = pl.num_programs(0)
    f8 = jnp.float8_e4m3fn

    @pl.when(g < 8)
    def _prep():
        mb = mb_ref[...]
        wn2 = (-2.0 * wt_ref[...]).astype(jnp.bfloat16)
        mbp = jnp.dot(
            mb.astype(jnp.bfloat16), wn2, preferred_element_type=jnp.float32
        )
        ksq = jnp.sum(mb * mb, axis=1, keepdims=True)
        ksq4 = ksq * 0.25
        hi = ksq4.astype(f8)
        r1 = ksq4 - hi.astype(jnp.float32)
        lo1 = r1.astype(f8)
        r2 = r1 - lo1.astype(jnp.float32)
        lo2 = r2.astype(f8)
        lo3 = (r2 - lo2.astype(jnp.float32)).astype(f8)
        kb = mb.shape[0]
        mba_s[pl.ds(g * kb, kb), :] = jnp.concatenate(
            [mbp.astype(f8), hi, lo1, lo2, lo3, jnp.zeros((kb, 60), f8)],
            axis=1,
        )

    @pl.when(g < 7)
    def _xt():
        blk = img_ref[...].astype(jnp.bfloat16)
        m = blk.reshape(768, 224)
        m1 = jnp.dot(
            m, e1_ref[...].astype(jnp.bfloat16),
            preferred_element_type=jnp.float32,
        ).astype(jnp.bfloat16)
        x = m1.reshape(8, 3, 4, 8, 8, 28)
        xb = x.transpose(1, 3, 4, 2, 0, 5).reshape(192, 896)
        xta_s[0:192, pl.ds(g * 896, 896)] = xb.astype(f8)
        f = jnp.dot(
            wt_ref[...].astype(jnp.bfloat16), xb,
            preferred_element_type=jnp.float32,
        )
        qsq_ref[...] = jnp.sum(f * f, axis=0, keepdims=True)

    @pl.when(g == 0)
    def _pad():
        q = xta_s.shape[1]
        xta_s[192:256, :] = jnp.concatenate(
            [jnp.full((4, q), 4.0, f8), jnp.zeros((60, q), f8)]
        )

    @pl.when(g >= 8)
    def _dist():
        k = g - 8
        aug = mba_s[pl.ds(k * 1024, 1024), :]
        d = jnp.dot(
            aug, xta_s[...], preferred_element_type=jnp.float32
        )
        part8 = jnp.min(d.reshape(128, 8, d.shape[1]), axis=0)

        @pl.when(k == 0)
        def _init():
            acc_ref[...] = part8

        @pl.when(k > 0)
        def _acc():
            acc_ref[...] = jnp.minimum(acc_ref[...], part8)

        @pl.when(g == ng - 1)
        def _fin():
            out_ref[...] = jnp.min(acc_ref[...], axis=0, keepdims=True)


def _post_kernel(dmin_ref, qsq_ref, r_ref, rt_ref, up_ref, mx_ref):
    g = dmin_ref[0] + qsq_ref[0]
    t = jnp.dot(r_ref[...], g, preferred_element_type=jnp.float32)
    up_ref[0] = jnp.dot(t, rt_ref[...], preferred_element_type=jnp.float32)
    mx_ref[0] = jnp.broadcast_to(jnp.max(g), (1, 128))


@functools.partial(jax.jit, static_argnames=())
def kernel(batch_images, W, memory_bank):
    B = batch_images.shape[0]
    Q = B * 28 * 28
    F = W.shape[1]
    K = memory_bank.shape[0]

    wt = W.T

    QB = 896
    KPB = K // 8
    qsq, dmin = pl.pallas_call(
        _main_kernel,
        grid=(8 + K // 1024,),
        in_specs=[
            pl.BlockSpec((B, 3, 32, 224), lambda g: (0, 0, jnp.minimum(g, 6), 0)),
            pl.BlockSpec((F, 192), lambda g: (0, 0)),
            pl.BlockSpec((224, 224), lambda g: (0, 0)),
            pl.BlockSpec((KPB, F), lambda g: (jnp.minimum(g, 7), 0)),
        ],
        out_specs=[
            pl.BlockSpec((1, QB), lambda g: (0, jnp.minimum(g, 6))),
            pl.BlockSpec((1, Q), lambda g: (0, 0)),
        ],
        out_shape=[
            jax.ShapeDtypeStruct((1, Q), jnp.float32),
            jax.ShapeDtypeStruct((1, Q), jnp.float32),
        ],
        scratch_shapes=[
            pltpu.VMEM((256, Q), jnp.float8_e4m3fn),
            pltpu.VMEM((K, 256), jnp.float8_e4m3fn),
            pltpu.VMEM((8, Q), jnp.float32),
        ],
    )(batch_images, wt, jnp.asarray(_E1), memory_bank)

    dmin_r = dmin.reshape(28, B, 28).transpose(1, 0, 2)
    qsq_r = qsq.reshape(28, B, 28).transpose(1, 0, 2)

    ups, mx = pl.pallas_call(
        _post_kernel,
        grid=(B,),
        in_specs=[
            pl.BlockSpec((1, 28, 28), lambda b: (b, 0, 0)),
            pl.BlockSpec((1, 28, 28), lambda b: (b, 0, 0)),
            pl.BlockSpec((224, 28), lambda b: (0, 0)),
            pl.BlockSpec((28, 224), lambda b: (0, 0)),
        ],
        out_specs=[
            pl.BlockSpec((1, 224, 224), lambda b: (b, 0, 0)),
            pl.BlockSpec((1, 1, 128), lambda b: (b, 0, 0)),
        ],
        out_shape=[
            jax.ShapeDtypeStruct((B, 224, 224), jnp.float32),
            jax.ShapeDtypeStruct((B, 1, 128), jnp.float32),
        ],
    )(dmin_r, qsq_r, jnp.asarray(_R_UP), jnp.asarray(_R_UP_T))

    image_scores = mx[:, 0, 0]
    return image_scores, ups
